# baseline (speedup 1.0000x reference)
import jax
import jax.numpy as jnp
from jax import lax
from jax.experimental import pallas as pl
from jax.experimental.pallas import tpu as pltpu
from jax.experimental.pallas import tpu_sc as plsc

D = 10000
L = 1000
P = 784
B = 4
NCLS = 10

NSC_CORES = 1
NWORK = NSC_CORES * 16
NGRP = 4
STRIP = NGRP * 16
CSC = NWORK * STRIP

TILE = 1024
TC_OFF = CSC // TILE
DTC = D - CSC
GRID = (DTC + TILE - 1) // TILE


def _tc_bundle_body(x_ref, v_ref, pos_ref, cw_ref, out_ref, acc_ref):
    i = pl.program_id(0)
    v = v_ref[...]
    b1 = v[L - 1:L, :]
    t = jnp.sum((v != b1).astype(jnp.int32), axis=0, keepdims=True)
    pos = pos_ref[...]
    s = jnp.sum(pos, axis=0, keepdims=True)
    xf = x_ref[...]
    idx = jnp.clip(jnp.round(xf * (L - 1)), 0, L - 1).astype(jnp.int32)
    rows = []
    for b in range(B):
        ib = idx[b, :].reshape(P, 1)
        s1 = jnp.sum(jnp.where(ib >= t, pos, 0.0), axis=0, keepdims=True)
        rows.append(b1 * (2.0 * s1 - s))
    bun = jnp.concatenate(rows, axis=0)
    enc = jnp.where(bun > 0, 1.0, -1.0)
    valid = D - (i + TC_OFF) * TILE
    lanes = jax.lax.broadcasted_iota(jnp.int32, (B, TILE), 1)
    enc = jnp.where(lanes < valid, enc, 0.0)
    part = jax.lax.dot_general(
        enc, cw_ref[...], (((1,), (1,)), ((), ())),
        preferred_element_type=jnp.float32)

    @pl.when(i == 0)
    def _():
        acc_ref[...] = part

    @pl.when(i > 0)
    def _():
        acc_ref[...] = acc_ref[...] + part

    @pl.when(i == GRID - 1)
    def _():
        out_ref[...] = acc_ref[...]


def _join_body(bun_ref, cw_ref, part_ref, out_ref):
    enc = jnp.where(bun_ref[...] > 0, 1.0, -1.0)
    out_ref[...] = part_ref[...] + jax.lax.dot_general(
        enc, cw_ref[...], (((1,), (1,)), ((), ())),
        preferred_element_type=jnp.float32)


def _sc_bundle_body(x_hbm, v_hbm, pos_hbm, out_hbm,
                    vbuf, pbuf, xbuf, idxbuf, accbuf, sem0, sem1, sem2):
    wid = lax.axis_index("s") * NSC_CORES + lax.axis_index("c")
    col0 = wid * STRIP

    hv = pltpu.async_copy(v_hbm.at[:, pl.ds(col0, STRIP)], vbuf, sem0)
    hp = pltpu.async_copy(pos_hbm.at[:, pl.ds(col0, STRIP)], pbuf, sem1)
    hi = pltpu.async_copy(x_hbm, xbuf, sem2)

    hi.wait()

    def q_step(i, _):
        y = xbuf[pl.ds(16 * i, 16)] * jnp.float32(L - 1)
        r = (y + jnp.float32(8388608.0)) - jnp.float32(8388608.0)
        yi = jnp.clip(r.astype(jnp.int32), 0, L - 1)
        idxbuf[pl.ds(16 * i, 16)] = yi
        return 0

    lax.fori_loop(0, (B * P) // 16, q_step, 0)

    hv.wait()
    b1 = [vbuf[L - 1, pl.ds(16 * j, 16)] for j in range(NGRP)]
    t = tuple(jnp.zeros((16,), jnp.int32) for _ in range(NGRP))

    def row_step(l, tc):
        out = list(tc)
        for u in range(2):
            for j in range(NGRP):
                v = vbuf[2 * l + u, pl.ds(16 * j, 16)]
                out[j] = out[j] + jnp.where(v != b1[j], 1, 0)
        return tuple(out)

    t = lax.fori_loop(0, L // 2, row_step, t)

    hp.wait()
    acc = tuple(jnp.zeros((16,), jnp.float32) for _ in range(B * NGRP))

    def pair_step(q, ac):
        out = list(ac)
        for u in range(2):
            p = 2 * q + u
            ivs = [plsc.load_gather(
                idxbuf, [jnp.full((16,), b * P + p, jnp.int32)])
                for b in range(B)]
            for j in range(NGRP):
                pv = pbuf[p, pl.ds(16 * j, 16)]
                for b in range(B):
                    sgn = jnp.where(ivs[b] >= t[j], 1.0, -1.0)
                    out[b * NGRP + j] = out[b * NGRP + j] + sgn * pv
        return tuple(out)

    acc = lax.fori_loop(0, P // 2, pair_step, acc)

    for b in range(B):
        for j in range(NGRP):
            accbuf[b, pl.ds(16 * j, 16)] = acc[b * NGRP + j] * b1[j]
    pltpu.sync_copy(accbuf, out_hbm.at[:, pl.ds(col0, STRIP)])


def kernel(x, position_weight, value_weight, classify_weight):
    flat = x.reshape(B, P)
    mesh = plsc.VectorSubcoreMesh(core_axis_name="c", subcore_axis_name="s",
                                  num_cores=NSC_CORES)
    bundled_sc = pl.kernel(
        _sc_bundle_body,
        out_type=jax.ShapeDtypeStruct((B, CSC), jnp.float32),
        mesh=mesh,
        compiler_params=pltpu.CompilerParams(use_tc_tiling_on_sc=False,
                                             needs_layout_passes=False),
        cost_estimate=pl.CostEstimate(
            flops=8 * CSC * (L + B * P),
            bytes_accessed=4 * CSC * (L + P) + 4 * B * P,
            transcendentals=0),
        scratch_types=[
            pltpu.VMEM((L, STRIP), jnp.float32),
            pltpu.VMEM((P, STRIP), jnp.float32),
            pltpu.VMEM((B * P,), jnp.float32),
            pltpu.VMEM((B * P,), jnp.int32),
            pltpu.VMEM((B, STRIP), jnp.float32),
            pltpu.SemaphoreType.DMA,
            pltpu.SemaphoreType.DMA,
            pltpu.SemaphoreType.DMA,
        ],
    )(flat.reshape(B * P), value_weight[:, :CSC], position_weight[:, :CSC])

    partial_tc = pl.pallas_call(
        _tc_bundle_body,
        grid=(GRID,),
        in_specs=[
            pl.BlockSpec((B, P), lambda i: (0, 0)),
            pl.BlockSpec((L, TILE), lambda i: (0, i + TC_OFF)),
            pl.BlockSpec((P, TILE), lambda i: (0, i + TC_OFF)),
            pl.BlockSpec((NCLS, TILE), lambda i: (0, i + TC_OFF)),
        ],
        out_specs=pl.BlockSpec((B, NCLS), lambda i: (0, 0)),
        out_shape=jax.ShapeDtypeStruct((B, NCLS), jnp.float32),
        scratch_shapes=[pltpu.VMEM((B, NCLS), jnp.float32)],
    )(flat, value_weight, position_weight, classify_weight)

    logit = pl.pallas_call(
        _join_body,
        grid=(1,),
        in_specs=[
            pl.BlockSpec((B, CSC), lambda i: (0, 0)),
            pl.BlockSpec((NCLS, CSC), lambda i: (0, 0)),
            pl.BlockSpec((B, NCLS), lambda i: (0, 0)),
        ],
        out_specs=pl.BlockSpec((B, NCLS), lambda i: (0, 0)),
        out_shape=jax.ShapeDtypeStruct((B, NCLS), jnp.float32),
    )(bundled_sc, classify_weight, partial_tc)
    return logit


# speedup vs baseline: 1.0031x; 1.0031x over previous
import jax
import jax.numpy as jnp
from jax import lax
from jax.experimental import pallas as pl
from jax.experimental.pallas import tpu as pltpu
from jax.experimental.pallas import tpu_sc as plsc

D = 10000
L = 1000
P = 784
B = 4
NCLS = 10

NSC_CORES = 1
NWORK = NSC_CORES * 16
NGRP = 4
STRIP = NGRP * 16
CSC = NWORK * STRIP

TILE = 1024
TC_OFF = CSC // TILE
DTC = D - CSC
GRID = (DTC + TILE - 1) // TILE


def _tc_bundle_body(x_ref, v_ref, pos_ref, cw_ref, out_ref, acc_ref):
    i = pl.program_id(0)
    v = v_ref[...]
    b1 = v[L - 1:L, :]
    t = jnp.sum((v != b1).astype(jnp.int32), axis=0, keepdims=True)
    pos = pos_ref[...]
    s = jnp.sum(pos, axis=0, keepdims=True)
    xf = x_ref[...]
    idx = jnp.clip(jnp.round(xf * (L - 1)), 0, L - 1).astype(jnp.int32)
    rows = []
    for b in range(B):
        ib = idx[b, :].reshape(P, 1)
        s1 = jnp.sum(jnp.where(ib >= t, pos, 0.0), axis=0, keepdims=True)
        rows.append(b1 * (2.0 * s1 - s))
    bun = jnp.concatenate(rows, axis=0)
    enc = jnp.where(bun > 0, 1.0, -1.0)
    valid = D - (i + TC_OFF) * TILE
    lanes = jax.lax.broadcasted_iota(jnp.int32, (B, TILE), 1)
    enc = jnp.where(lanes < valid, enc, 0.0)
    part = jax.lax.dot_general(
        enc, cw_ref[...], (((1,), (1,)), ((), ())),
        preferred_element_type=jnp.float32)

    @pl.when(i == 0)
    def _():
        acc_ref[...] = part

    @pl.when(i > 0)
    def _():
        acc_ref[...] = acc_ref[...] + part

    @pl.when(i == GRID - 1)
    def _():
        out_ref[...] = acc_ref[...]


def _join_body(bun_ref, cw_ref, part_ref, out_ref):
    enc = jnp.where(bun_ref[...] > 0, 1.0, -1.0)
    out_ref[...] = part_ref[...] + jax.lax.dot_general(
        enc, cw_ref[...], (((1,), (1,)), ((), ())),
        preferred_element_type=jnp.float32)


def _sc_bundle_body(x_hbm, v_hbm, pos_hbm, out_hbm,
                    vbuf, pbuf, xbuf, idxbuf, accbuf, sem0, sem1, sem2):
    wid = lax.axis_index("s") * NSC_CORES + lax.axis_index("c")
    col0 = wid * STRIP

    hv = pltpu.async_copy(v_hbm.at[:, pl.ds(col0, STRIP)], vbuf, sem0)
    hp = pltpu.async_copy(pos_hbm.at[:, pl.ds(col0, STRIP)], pbuf, sem1)
    hi = pltpu.async_copy(x_hbm, xbuf, sem2)

    hi.wait()

    def q_step(i, _):
        y = xbuf[pl.ds(16 * i, 16)] * jnp.float32(L - 1)
        r = (y + jnp.float32(8388608.0)) - jnp.float32(8388608.0)
        yi = jnp.clip(r.astype(jnp.int32), 0, L - 1)
        idxbuf[pl.ds(16 * i, 16)] = yi
        return 0

    lax.fori_loop(0, (B * P) // 16, q_step, 0)

    hv.wait()
    b1 = [vbuf[L - 1, pl.ds(16 * j, 16)] for j in range(NGRP)]
    t = tuple(jnp.zeros((16,), jnp.int32) for _ in range(NGRP))

    def row_step(l, tc):
        out = list(tc)
        for u in range(2):
            for j in range(NGRP):
                v = vbuf[2 * l + u, pl.ds(16 * j, 16)]
                out[j] = out[j] + jnp.where(v != b1[j], 1, 0)
        return tuple(out)

    t = lax.fori_loop(0, L // 2, row_step, t)

    hp.wait()
    acc = tuple(jnp.zeros((16,), jnp.float32) for _ in range(B * NGRP))

    def pair_step(q, ac):
        out = list(ac)
        for u in range(2):
            p = 2 * q + u
            ivs = [plsc.load_gather(
                idxbuf, [jnp.full((16,), b * P + p, jnp.int32)])
                for b in range(B)]
            for j in range(NGRP):
                pv = pbuf[p, pl.ds(16 * j, 16)]
                for b in range(B):
                    sgn = jnp.where(ivs[b] >= t[j], 1.0, -1.0)
                    out[b * NGRP + j] = out[b * NGRP + j] + sgn * pv
        return tuple(out)

    acc = lax.fori_loop(0, P // 2, pair_step, acc)

    for b in range(B):
        for j in range(NGRP):
            accbuf[b, pl.ds(16 * j, 16)] = acc[b * NGRP + j] * b1[j]
    pltpu.sync_copy(accbuf, out_hbm.at[:, pl.ds(col0, STRIP)])


def kernel(x, position_weight, value_weight, classify_weight):
    flat = x.reshape(B, P)
    mesh = plsc.VectorSubcoreMesh(core_axis_name="c", subcore_axis_name="s",
                                  num_cores=NSC_CORES)
    bundled_sc = pl.kernel(
        _sc_bundle_body,
        out_type=jax.ShapeDtypeStruct((B, CSC), jnp.float32),
        mesh=mesh,
        compiler_params=pltpu.CompilerParams(use_tc_tiling_on_sc=False,
                                             needs_layout_passes=False),
        scratch_types=[
            pltpu.VMEM((L, STRIP), jnp.float32),
            pltpu.VMEM((P, STRIP), jnp.float32),
            pltpu.VMEM((B * P,), jnp.float32),
            pltpu.VMEM((B * P,), jnp.int32),
            pltpu.VMEM((B, STRIP), jnp.float32),
            pltpu.SemaphoreType.DMA,
            pltpu.SemaphoreType.DMA,
            pltpu.SemaphoreType.DMA,
        ],
    )(flat.reshape(B * P), value_weight[:, :CSC], position_weight[:, :CSC])

    partial_tc = pl.pallas_call(
        _tc_bundle_body,
        grid=(GRID,),
        in_specs=[
            pl.BlockSpec((B, P), lambda i: (0, 0)),
            pl.BlockSpec((L, TILE), lambda i: (0, i + TC_OFF)),
            pl.BlockSpec((P, TILE), lambda i: (0, i + TC_OFF)),
            pl.BlockSpec((NCLS, TILE), lambda i: (0, i + TC_OFF)),
        ],
        out_specs=pl.BlockSpec((B, NCLS), lambda i: (0, 0)),
        out_shape=jax.ShapeDtypeStruct((B, NCLS), jnp.float32),
        scratch_shapes=[pltpu.VMEM((B, NCLS), jnp.float32)],
    )(flat, value_weight, position_weight, classify_weight)

    logit = pl.pallas_call(
        _join_body,
        grid=(1,),
        in_specs=[
            pl.BlockSpec((B, CSC), lambda i: (0, 0)),
            pl.BlockSpec((NCLS, CSC), lambda i: (0, 0)),
            pl.BlockSpec((B, NCLS), lambda i: (0, 0)),
        ],
        out_specs=pl.BlockSpec((B, NCLS), lambda i: (0, 0)),
        out_shape=jax.ShapeDtypeStruct((B, NCLS), jnp.float32),
    )(bundled_sc, classify_weight, partial_tc)
    return logit
